# baseline (speedup 1.0000x reference)
import functools

import jax
import jax.numpy as jnp
from jax import lax
from jax.experimental import pallas as pl
from jax.experimental.pallas import tpu as pltpu
from jax.experimental.pallas import tpu_sc as plsc

EMBED = 64
PAIR = 2 * EMBED
LANES = 16
NCORES = 2
NSUB = 16
NWORKERS = NCORES * NSUB
IDX_CHUNK = 128
ROUND = 256


def _sc_body(bpw, tid_hbm, cid_hbm, table_in, table_out, o_hbm,
             ids_t, ids_c, pid_t, pid_c, rows_t, rows_c, tpose, out_v, sem):
    wid = lax.axis_index("s") * NCORES + lax.axis_index("c")
    base = wid * bpw

    pltpu.sync_copy(tid_hbm.at[pl.ds(base, bpw)], ids_t)
    pltpu.sync_copy(cid_hbm.at[pl.ds(base, bpw)], ids_c)

    for j in range(bpw // LANES):
        sl = pl.ds(j * LANES, LANES)
        pid_t[sl] = lax.shift_right_logical(ids_t[sl], 1)
        pid_c[sl] = lax.shift_right_logical(ids_c[sl], 1)

    iota = lax.iota(jnp.int32, LANES)
    one = jnp.int32(1)

    for h in range(bpw // ROUND):
        hbase = h * ROUND
        copies = []
        for j in range(ROUND // IDX_CHUNK):
            off = j * IDX_CHUNK
            copies.append(pltpu.async_copy(
                table_in.at[pid_t.at[pl.ds(hbase + off, IDX_CHUNK)]],
                rows_t.at[pl.ds(off, IDX_CHUNK)], sem))
            copies.append(pltpu.async_copy(
                table_out.at[pid_c.at[pl.ds(hbase + off, IDX_CHUNK)]],
                rows_c.at[pl.ds(off, IDX_CHUNK)], sem))
        for cp in copies:
            cp.wait()

        def group(g, carry):
            rbase = g * LANES
            idt16 = ids_t[pl.ds(hbase + rbase, LANES)]
            idc16 = ids_c[pl.ds(hbase + rbase, LANES)]
            for r in range(LANES):
                row = rbase + r
                pt = jnp.bitwise_and(idt16[r], one)
                pc = jnp.bitwise_and(idc16[r], one)
                acc = None
                for c in range(EMBED // LANES):
                    tlo = rows_t[row, pl.ds(c * LANES, LANES)]
                    thi = rows_t[row, pl.ds(EMBED + c * LANES, LANES)]
                    clo = rows_c[row, pl.ds(c * LANES, LANES)]
                    chi = rows_c[row, pl.ds(EMBED + c * LANES, LANES)]
                    tsel = jnp.where(pt == 1, thi, tlo)
                    csel = jnp.where(pc == 1, chi, clo)
                    prod = tsel * csel
                    acc = prod if acc is None else acc + prod
                plsc.store_scatter(tpose, [iota * LANES + r], acc)
            colsum = tpose[pl.ds(0, LANES)]
            for l in range(1, LANES):
                colsum = colsum + tpose[pl.ds(l * LANES, LANES)]
            out_v[pl.ds(hbase + rbase, LANES)] = colsum
            return carry

        lax.fori_loop(0, ROUND // LANES, group, 0)

    pltpu.sync_copy(out_v, o_hbm.at[pl.ds(base, bpw)])


def kernel(target_ids, context_ids, in_embed, out_embed):
    batch = target_ids.shape[0]
    vocab = in_embed.shape[0]
    bpw = batch // NWORKERS
    mesh = plsc.VectorSubcoreMesh(core_axis_name="c", subcore_axis_name="s")
    f = pl.kernel(
        functools.partial(_sc_body, bpw),
        out_type=jax.ShapeDtypeStruct((batch,), jnp.float32),
        mesh=mesh,
        scratch_types=[
            pltpu.VMEM((bpw,), jnp.int32),
            pltpu.VMEM((bpw,), jnp.int32),
            pltpu.VMEM((bpw,), jnp.int32),
            pltpu.VMEM((bpw,), jnp.int32),
            pltpu.VMEM((ROUND, PAIR), jnp.float32),
            pltpu.VMEM((ROUND, PAIR), jnp.float32),
            pltpu.VMEM((LANES * LANES,), jnp.float32),
            pltpu.VMEM((bpw,), jnp.float32),
            pltpu.SemaphoreType.DMA,
        ],
        compiler_params=pltpu.CompilerParams(needs_layout_passes=False,
                                             use_tc_tiling_on_sc=True),
    )
    tbl_in = in_embed.reshape(vocab // 2, PAIR)
    tbl_out = out_embed.reshape(vocab // 2, PAIR)
    return f(target_ids.astype(jnp.int32), context_ids.astype(jnp.int32),
             tbl_in, tbl_out)


# speedup vs baseline: 1.0012x; 1.0012x over previous
import functools

import jax
import jax.numpy as jnp
from jax import lax
from jax.experimental import pallas as pl
from jax.experimental.pallas import tpu as pltpu
from jax.experimental.pallas import tpu_sc as plsc

EMBED = 64
LANES = 16
NCORES = 2
NSUB = 16
NWORKERS = NCORES * NSUB
IDX_CHUNK = 128

_PARAMS = pltpu.CompilerParams(needs_layout_passes=False,
                               use_tc_tiling_on_sc=False)


def _gather_body(bpw, nchunk, tid_hbm, table, rows_hbm,
                 idx, rows_v, sem):
    wid = lax.axis_index("s") * NCORES + lax.axis_index("c")
    base = wid * bpw

    for j in range(nchunk):
        pltpu.sync_copy(tid_hbm.at[pl.ds(base + j * IDX_CHUNK, IDX_CHUNK)],
                        idx.at[j])
    copies = []
    for j in range(nchunk):
        copies.append(pltpu.async_copy(
            table.at[idx.at[j]],
            rows_v.at[pl.ds(j * IDX_CHUNK, IDX_CHUNK)], sem))
    for cp in copies:
        cp.wait()
    pltpu.sync_copy(rows_v, rows_hbm.at[pl.ds(base, bpw), :])


def _dot_body(bpw, nchunk, cid_hbm, table, rows_in_hbm, o_hbm,
              idx, rows_t, rows_c, tpose, out_v, sem):
    wid = lax.axis_index("s") * NCORES + lax.axis_index("c")
    base = wid * bpw

    pltpu.sync_copy(rows_in_hbm.at[pl.ds(base, bpw), :], rows_t)
    for j in range(nchunk):
        pltpu.sync_copy(cid_hbm.at[pl.ds(base + j * IDX_CHUNK, IDX_CHUNK)],
                        idx.at[j])
    copies = []
    for j in range(nchunk):
        copies.append(pltpu.async_copy(
            table.at[idx.at[j]],
            rows_c.at[pl.ds(j * IDX_CHUNK, IDX_CHUNK)], sem))
    for cp in copies:
        cp.wait()

    iota = lax.iota(jnp.int32, LANES)

    def group(g, carry):
        rbase = g * LANES
        for r in range(LANES):
            row = rbase + r
            acc = rows_t[row, pl.ds(0, LANES)] * rows_c[row, pl.ds(0, LANES)]
            for c in range(1, EMBED // LANES):
                acc = acc + (rows_t[row, pl.ds(c * LANES, LANES)] *
                             rows_c[row, pl.ds(c * LANES, LANES)])
            plsc.store_scatter(tpose, [iota * LANES + r], acc)
        colsum = tpose[pl.ds(0, LANES)]
        for l in range(1, LANES):
            colsum = colsum + tpose[pl.ds(l * LANES, LANES)]
        out_v[pl.ds(rbase, LANES)] = colsum
        return carry

    lax.fori_loop(0, bpw // LANES, group, 0)
    pltpu.sync_copy(out_v, o_hbm.at[pl.ds(base, bpw)])


def kernel(target_ids, context_ids, in_embed, out_embed):
    batch = target_ids.shape[0]
    bpw = batch // NWORKERS
    nchunk = bpw // IDX_CHUNK
    mesh = plsc.VectorSubcoreMesh(core_axis_name="c", subcore_axis_name="s")

    gather_in = pl.kernel(
        functools.partial(_gather_body, bpw, nchunk),
        out_type=jax.ShapeDtypeStruct((batch, EMBED), jnp.float32),
        mesh=mesh,
        scratch_types=[
            pltpu.VMEM((nchunk, IDX_CHUNK), jnp.int32),
            pltpu.VMEM((bpw, EMBED), jnp.float32),
            pltpu.SemaphoreType.DMA,
        ],
        compiler_params=_PARAMS,
    )
    dot_out = pl.kernel(
        functools.partial(_dot_body, bpw, nchunk),
        out_type=jax.ShapeDtypeStruct((batch,), jnp.float32),
        mesh=mesh,
        scratch_types=[
            pltpu.VMEM((nchunk, IDX_CHUNK), jnp.int32),
            pltpu.VMEM((bpw, EMBED), jnp.float32),
            pltpu.VMEM((bpw, EMBED), jnp.float32),
            pltpu.VMEM((LANES * LANES,), jnp.float32),
            pltpu.VMEM((bpw,), jnp.float32),
            pltpu.SemaphoreType.DMA,
        ],
        compiler_params=_PARAMS,
    )
    rows_in = gather_in(target_ids.astype(jnp.int32), in_embed)
    return dot_out(context_ids.astype(jnp.int32), out_embed, rows_in)
